# Initial kernel scaffold; baseline (speedup 1.0000x reference)
import jax
import jax.numpy as jnp
from jax.experimental import pallas as pl


def kernel(feat, edge_index, biclique_mask, W, attn_param):
    raise NotImplementedError("write your pallas kernel here")


# speedup vs baseline: 18.7830x; 18.7830x over previous
import functools

import jax
import jax.numpy as jnp
from jax import lax
from jax.experimental import pallas as pl
from jax.experimental.pallas import tpu as pltpu
from jax.experimental.pallas import tpu_sc as plsc

N = 10000
E = 320000
D = 128
WD = 144
NC = 2
NS = 16
NW = NC * NS
K = 125
NB = E // (NW * K)
ROWS_PER_TILE = N // NS


def _prep_body(feat_ref, mask_ref, wt_ref, attn_ref, g_ref):
    fm = feat_ref[...] * mask_ref[...]
    h = jnp.dot(fm, wt_ref[...], preferred_element_type=jnp.float32)
    s = jnp.dot(h, attn_ref[...], preferred_element_type=jnp.float32)
    t = jnp.where(s > 0.0, s, s * 0.01)
    p = jnp.exp(t - jnp.max(t))
    g_ref[...] = jnp.concatenate([h * p, jnp.broadcast_to(p, (N, WD - D))], axis=1)


_prep = pl.pallas_call(
    _prep_body,
    out_shape=jax.ShapeDtypeStruct((N, WD), jnp.float32),
)


def _final_body(acc_ref, o_ref):
    num = acc_ref[0, :, :D] + acc_ref[1, :, :D]
    den = acc_ref[0, :, D:D + 1] + acc_ref[1, :, D:D + 1]
    o_ref[...] = jnp.maximum(num / jnp.where(den > 0.0, den, 1.0), 0.0)


_final = pl.pallas_call(
    _final_body,
    out_shape=jax.ShapeDtypeStruct((N, D), jnp.float32),
)


@functools.partial(
    pl.kernel,
    out_type=jax.ShapeDtypeStruct((NC, N, WD), jnp.float32),
    mesh=plsc.VectorSubcoreMesh(core_axis_name="c", subcore_axis_name="s"),
    scratch_types=[
        pltpu.VMEM((NB, K), jnp.int32),
        pltpu.VMEM((NB, K), jnp.int32),
        pltpu.VMEM((K, WD), jnp.float32),
        pltpu.VMEM_SHARED((N, WD), jnp.float32),
        pltpu.SemaphoreType.DMA,
    ],
    compiler_params=pltpu.CompilerParams(use_tc_tiling_on_sc=False),
)
def _edge_kernel(g_hbm, src_hbm, dst_hbm, out_hbm, sidx, didx, rows, acc, sem):
    cid = lax.axis_index("c")
    sid = lax.axis_index("s")
    w = cid * NS + sid

    @pl.loop(0, K)
    def _zero_rows(i):
        @pl.loop(0, WD, step=16)
        def _zero_lane(c):
            rows[i, pl.ds(c, 16)] = jnp.zeros((16,), jnp.float32)

    @pl.loop(0, ROWS_PER_TILE, step=K)
    def _zero_acc(r):
        pltpu.sync_copy(rows, acc.at[pl.ds(sid * ROWS_PER_TILE + r, K)])

    pltpu.sync_copy(src_hbm.at[w], sidx)
    pltpu.sync_copy(dst_hbm.at[w], didx)

    plsc.subcore_barrier()

    @pl.loop(0, NB)
    def _block(j):
        pltpu.async_copy(g_hbm.at[sidx.at[j]], rows, sem).wait()
        pltpu.sync_copy(rows, acc.at[didx.at[j]], add=True)

    plsc.subcore_barrier()
    pltpu.sync_copy(
        acc.at[pl.ds(sid * ROWS_PER_TILE, ROWS_PER_TILE)],
        out_hbm.at[cid, pl.ds(sid * ROWS_PER_TILE, ROWS_PER_TILE)],
    )


def kernel(feat, edge_index, biclique_mask, W, attn_param):
    src = edge_index[0].astype(jnp.int32).reshape(NW, NB, K)
    dst = edge_index[1].astype(jnp.int32).reshape(NW, NB, K)
    g = _prep(feat, biclique_mask.reshape(1, D), W.T, attn_param)
    acc = _edge_kernel(g, src, dst)
    return _final(acc)


# speedup vs baseline: 20.3098x; 1.0813x over previous
import functools

import jax
import jax.numpy as jnp
from jax import lax
from jax.experimental import pallas as pl
from jax.experimental.pallas import tpu as pltpu
from jax.experimental.pallas import tpu_sc as plsc

N = 10000
E = 320000
D = 128
WD = 144
NC = 2
NS = 16
NW = NC * NS
K = 125
NB = E // (NW * K)
IDXC = 16
ROWS_PER_TILE = N // NS


def _prep_body(feat_ref, mask_ref, w_ref, attn_ref, g_ref):
    fm = feat_ref[...] * mask_ref[...]
    h = lax.dot_general(fm, w_ref[...], (((1,), (1,)), ((), ())),
                        preferred_element_type=jnp.float32)
    s = jnp.dot(h, attn_ref[...], preferred_element_type=jnp.float32)
    t = jnp.where(s > 0.0, s, s * 0.01)
    p = jnp.exp(t - jnp.max(t))
    g_ref[...] = jnp.concatenate([h * p, jnp.broadcast_to(p, (N, WD - D))], axis=1)


_prep = pl.pallas_call(
    _prep_body,
    out_shape=jax.ShapeDtypeStruct((N, WD), jnp.float32),
)


def _final_body(acc_ref, o_ref):
    num = acc_ref[0, :, :D] + acc_ref[1, :, :D]
    den = acc_ref[0, :, D:D + 1] + acc_ref[1, :, D:D + 1]
    o_ref[...] = jnp.maximum(num / jnp.where(den > 0.0, den, 1.0), 0.0)


_final = pl.pallas_call(
    _final_body,
    out_shape=jax.ShapeDtypeStruct((N, D), jnp.float32),
)


@functools.partial(
    pl.kernel,
    out_type=jax.ShapeDtypeStruct((NC, N, WD), jnp.float32),
    mesh=plsc.VectorSubcoreMesh(core_axis_name="c", subcore_axis_name="s"),
    scratch_types=[
        pltpu.VMEM((IDXC, K), jnp.int32),
        pltpu.VMEM((IDXC, K), jnp.int32),
        pltpu.VMEM((K, WD), jnp.float32),
        pltpu.VMEM((K, WD), jnp.float32),
        pltpu.VMEM_SHARED((N, WD), jnp.float32),
        pltpu.SemaphoreType.DMA,
        pltpu.SemaphoreType.DMA,
    ],
    compiler_params=pltpu.CompilerParams(use_tc_tiling_on_sc=False),
)
def _edge_kernel(g_hbm, src_hbm, dst_hbm, out_hbm, sidx, didx,
                 rows0, rows1, acc, sem0, sem1):
    cid = lax.axis_index("c")
    sid = lax.axis_index("s")
    w = cid * NS + sid

    @pl.loop(0, K)
    def _zero_rows(i):
        @pl.loop(0, WD, step=16)
        def _zero_lane(c):
            rows0[i, pl.ds(c, 16)] = jnp.zeros((16,), jnp.float32)

    @pl.loop(0, ROWS_PER_TILE, step=K)
    def _zero_acc(r):
        pltpu.sync_copy(rows0, acc.at[pl.ds(sid * ROWS_PER_TILE + r, K)])

    plsc.subcore_barrier()

    @pl.loop(0, NB // IDXC)
    def _chunk(c):
        pltpu.sync_copy(src_hbm.at[w].at[pl.ds(c * IDXC, IDXC)], sidx)
        pltpu.sync_copy(dst_hbm.at[w].at[pl.ds(c * IDXC, IDXC)], didx)

        @pl.loop(0, IDXC, step=2)
        def _blk(i):
            c0 = pltpu.async_copy(g_hbm.at[sidx.at[i]], rows0, sem0)
            c1 = pltpu.async_copy(g_hbm.at[sidx.at[i + 1]], rows1, sem1)
            c0.wait()
            pltpu.sync_copy(rows0, acc.at[didx.at[i]], add=True)
            c1.wait()
            pltpu.sync_copy(rows1, acc.at[didx.at[i + 1]], add=True)

    plsc.subcore_barrier()
    pltpu.sync_copy(
        acc.at[pl.ds(sid * ROWS_PER_TILE, ROWS_PER_TILE)],
        out_hbm.at[cid, pl.ds(sid * ROWS_PER_TILE, ROWS_PER_TILE)],
    )


def kernel(feat, edge_index, biclique_mask, W, attn_param):
    src = edge_index[0].astype(jnp.int32).reshape(NW, NB, K)
    dst = edge_index[1].astype(jnp.int32).reshape(NW, NB, K)
    g = _prep(feat, biclique_mask.reshape(1, D), W.T, attn_param)
    acc = _edge_kernel(g, src, dst)
    return _final(acc)


# speedup vs baseline: 20.4710x; 1.0079x over previous
import functools

import jax
import jax.numpy as jnp
from jax import lax
from jax.experimental import pallas as pl
from jax.experimental.pallas import tpu as pltpu
from jax.experimental.pallas import tpu_sc as plsc

N = 10000
E = 320000
D = 128
WD = 144
NC = 2
NS = 16
NW = NC * NS
K = 125
NB = E // (NW * K)
IDXC = 16
ROWS_PER_TILE = N // NS


def _prep_body(feat_ref, mask_ref, w_ref, attn_ref, g_ref):
    fm = feat_ref[...] * mask_ref[...]
    h = lax.dot_general(fm, w_ref[...], (((1,), (1,)), ((), ())),
                        preferred_element_type=jnp.float32)
    s = jnp.dot(h, attn_ref[...], preferred_element_type=jnp.float32)
    t = jnp.where(s > 0.0, s, s * 0.01)
    p = jnp.exp(t - jnp.max(t))
    g_ref[...] = jnp.concatenate([h * p, jnp.broadcast_to(p, (N, WD - D))], axis=1)


_prep = pl.pallas_call(
    _prep_body,
    out_shape=jax.ShapeDtypeStruct((N, WD), jnp.float32),
)


def _final_body(acc_ref, o_ref):
    num = acc_ref[0, :, :D] + acc_ref[1, :, :D]
    den = acc_ref[0, :, D:D + 1] + acc_ref[1, :, D:D + 1]
    o_ref[...] = jnp.maximum(num / jnp.where(den > 0.0, den, 1.0), 0.0)


_final = pl.pallas_call(
    _final_body,
    out_shape=jax.ShapeDtypeStruct((N, D), jnp.float32),
)


@functools.partial(
    pl.kernel,
    out_type=jax.ShapeDtypeStruct((NC, N, WD), jnp.float32),
    mesh=plsc.VectorSubcoreMesh(core_axis_name="c", subcore_axis_name="s"),
    scratch_types=[
        pltpu.VMEM((IDXC, K), jnp.int32),
        pltpu.VMEM((IDXC, K), jnp.int32),
        pltpu.VMEM((K, WD), jnp.float32),
        pltpu.VMEM((K, WD), jnp.float32),
        pltpu.VMEM_SHARED((N, WD), jnp.float32),
        pltpu.SemaphoreType.DMA,
        pltpu.SemaphoreType.DMA,
    ],
    compiler_params=pltpu.CompilerParams(use_tc_tiling_on_sc=False),
)
def _edge_kernel(g_hbm, src_hbm, dst_hbm, out_hbm, sidx, didx, rows0, rows1,
                 acc, sem0, sem1):
    cid = lax.axis_index("c")
    sid = lax.axis_index("s")
    w = cid * NS + sid

    @pl.loop(0, K)
    def _zero_rows(i):
        @pl.loop(0, WD, step=16)
        def _zero_lane(c):
            rows0[i, pl.ds(c, 16)] = jnp.zeros((16,), jnp.float32)

    @pl.loop(0, ROWS_PER_TILE, step=K)
    def _zero_acc(r):
        pltpu.sync_copy(rows0, acc.at[pl.ds(sid * ROWS_PER_TILE + r, K)])

    plsc.subcore_barrier()

    @pl.loop(0, NB // IDXC)
    def _chunk(c):
        pltpu.sync_copy(src_hbm.at[w].at[pl.ds(c * IDXC, IDXC)], sidx)
        pltpu.sync_copy(dst_hbm.at[w].at[pl.ds(c * IDXC, IDXC)], didx)

        @pl.loop(0, IDXC, step=2)
        def _blk(i):
            c0 = pltpu.async_copy(g_hbm.at[sidx.at[i]], rows0, sem0)
            c1 = pltpu.async_copy(g_hbm.at[sidx.at[i + 1]], rows1, sem1)
            c0.wait()
            pltpu.sync_copy(rows0, acc.at[didx.at[i]], add=True)
            c1.wait()
            pltpu.sync_copy(rows1, acc.at[didx.at[i + 1]], add=True)

    plsc.subcore_barrier()
    pltpu.sync_copy(
        acc.at[pl.ds(sid * ROWS_PER_TILE, ROWS_PER_TILE)],
        out_hbm.at[cid, pl.ds(sid * ROWS_PER_TILE, ROWS_PER_TILE)],
    )


def kernel(feat, edge_index, biclique_mask, W, attn_param):
    src = edge_index[0].astype(jnp.int32).reshape(NW, NB, K)
    dst = edge_index[1].astype(jnp.int32).reshape(NW, NB, K)
    g = _prep(feat, biclique_mask.reshape(1, D), W, attn_param)
    acc = _edge_kernel(g, src, dst)
    return _final(acc)


# speedup vs baseline: 23.5793x; 1.1518x over previous
import functools

import jax
import jax.numpy as jnp
from jax import lax
from jax.experimental import pallas as pl
from jax.experimental.pallas import tpu as pltpu
from jax.experimental.pallas import tpu_sc as plsc

N = 10000
E = 320000
D = 128
DP = 16
NC = 2
NS = 16
NW = NC * NS
K = 125
NB = E // (NW * K)
IDXC = 16
ROWS_PER_TILE = N // NS


def _prep_body(feat_ref, mask_ref, w_ref, attn_ref, gh_ref, p_ref):
    fm = feat_ref[...] * mask_ref[...]
    h = lax.dot_general(fm, w_ref[...], (((1,), (1,)), ((), ())),
                        preferred_element_type=jnp.float32)
    s = jnp.dot(h, attn_ref[...], preferred_element_type=jnp.float32)
    t = jnp.where(s > 0.0, s, s * 0.01)
    p = jnp.exp(t - jnp.max(t))
    gh_ref[...] = h * p
    p_ref[...] = jnp.broadcast_to(p, (N, DP))


_prep = pl.pallas_call(
    _prep_body,
    out_shape=(jax.ShapeDtypeStruct((N, D), jnp.float32),
               jax.ShapeDtypeStruct((N, DP), jnp.float32)),
)


def _final_body(acc_ref, accp_ref, o_ref):
    num = acc_ref[0] + acc_ref[1]
    den = accp_ref[0, :, :1] + accp_ref[1, :, :1]
    o_ref[...] = jnp.maximum(num / jnp.where(den > 0.0, den, 1.0), 0.0)


_final = pl.pallas_call(
    _final_body,
    out_shape=jax.ShapeDtypeStruct((N, D), jnp.float32),
)


@functools.partial(
    pl.kernel,
    out_type=(jax.ShapeDtypeStruct((NC, N, D), jnp.float32),
              jax.ShapeDtypeStruct((NC, N, DP), jnp.float32)),
    mesh=plsc.VectorSubcoreMesh(core_axis_name="c", subcore_axis_name="s"),
    scratch_types=[
        pltpu.VMEM((IDXC, K), jnp.int32),
        pltpu.VMEM((IDXC, K), jnp.int32),
        pltpu.VMEM((K, D), jnp.float32),
        pltpu.VMEM((K, D), jnp.float32),
        pltpu.VMEM((K, DP), jnp.float32),
        pltpu.VMEM((K, DP), jnp.float32),
        pltpu.VMEM_SHARED((N, D), jnp.float32),
        pltpu.VMEM_SHARED((N, DP), jnp.float32),
        pltpu.SemaphoreType.DMA,
        pltpu.SemaphoreType.DMA,
        pltpu.SemaphoreType.DMA,
        pltpu.SemaphoreType.DMA,
    ],
    compiler_params=pltpu.CompilerParams(use_tc_tiling_on_sc=False),
)
def _edge_kernel(g_hbm, p_hbm, e_hbm, out_hbm, outp_hbm, sidx, didx,
                 a0, a1, b0, b1, acc, accp, gsem0, gsem1, ssem0, ssem1):
    cid = lax.axis_index("c")
    sid = lax.axis_index("s")
    w = cid * NS + sid

    @pl.loop(0, K)
    def _zero_rows(i):
        @pl.loop(0, D, step=16)
        def _zero_lane(c):
            a0[i, pl.ds(c, 16)] = jnp.zeros((16,), jnp.float32)
        b0[i, pl.ds(0, DP)] = jnp.zeros((16,), jnp.float32)

    @pl.loop(0, ROWS_PER_TILE, step=K)
    def _zero_acc(r):
        pltpu.sync_copy(a0, acc.at[pl.ds(sid * ROWS_PER_TILE + r, K)])
        pltpu.sync_copy(b0, accp.at[pl.ds(sid * ROWS_PER_TILE + r, K)])

    plsc.subcore_barrier()

    def _gather(i, abuf, bbuf, sem):
        pltpu.async_copy(g_hbm.at[sidx.at[i]], abuf, sem)
        pltpu.async_copy(p_hbm.at[sidx.at[i]], bbuf, sem)

    def _gather_wait(i, abuf, bbuf, sem):
        pltpu.make_async_copy(g_hbm.at[sidx.at[i]], abuf, sem).wait()
        pltpu.make_async_copy(p_hbm.at[sidx.at[i]], bbuf, sem).wait()

    @pl.loop(0, NB // IDXC)
    def _chunk(c):
        pltpu.sync_copy(e_hbm.at[0, w, pl.ds(c * IDXC, IDXC)], sidx)
        pltpu.sync_copy(e_hbm.at[1, w, pl.ds(c * IDXC, IDXC)], didx)

        _gather(0, a0, b0, gsem0)
        _gather(1, a1, b1, gsem1)

        @pl.loop(0, IDXC - 2, step=2)
        def _blk(i):
            _gather_wait(i, a0, b0, gsem0)
            s0a = pltpu.async_copy(a0, acc.at[didx.at[i]], ssem0, add=True)
            s0b = pltpu.async_copy(b0, accp.at[didx.at[i]], ssem0, add=True)
            _gather_wait(i + 1, a1, b1, gsem1)
            s1a = pltpu.async_copy(a1, acc.at[didx.at[i + 1]], ssem1, add=True)
            s1b = pltpu.async_copy(b1, accp.at[didx.at[i + 1]], ssem1, add=True)
            s0a.wait()
            s0b.wait()
            _gather(i + 2, a0, b0, gsem0)
            s1a.wait()
            s1b.wait()
            _gather(i + 3, a1, b1, gsem1)

        _gather_wait(IDXC - 2, a0, b0, gsem0)
        pltpu.sync_copy(a0, acc.at[didx.at[IDXC - 2]], add=True)
        pltpu.sync_copy(b0, accp.at[didx.at[IDXC - 2]], add=True)
        _gather_wait(IDXC - 1, a1, b1, gsem1)
        pltpu.sync_copy(a1, acc.at[didx.at[IDXC - 1]], add=True)
        pltpu.sync_copy(b1, accp.at[didx.at[IDXC - 1]], add=True)

    plsc.subcore_barrier()
    row0 = sid * ROWS_PER_TILE
    pltpu.sync_copy(acc.at[pl.ds(row0, ROWS_PER_TILE)],
                    out_hbm.at[cid, pl.ds(row0, ROWS_PER_TILE)])
    pltpu.sync_copy(accp.at[pl.ds(row0, ROWS_PER_TILE)],
                    outp_hbm.at[cid, pl.ds(row0, ROWS_PER_TILE)])


def kernel(feat, edge_index, biclique_mask, W, attn_param):
    e4 = edge_index.astype(jnp.int32).reshape(2, NW, NB, K)
    gh, p16 = _prep(feat, biclique_mask.reshape(1, D), W, attn_param)
    acc, accp = _edge_kernel(gh, p16, e4)
    return _final(acc, accp)


# speedup vs baseline: 23.8871x; 1.0131x over previous
import functools

import jax
import jax.numpy as jnp
from jax import lax
from jax.experimental import pallas as pl
from jax.experimental.pallas import tpu as pltpu
from jax.experimental.pallas import tpu_sc as plsc

N = 10000
E = 320000
D = 128
L = 16
NR = N // L
NC = 2
NS = 16
NW = NC * NS
K = 80
NB = E // (NW * K)
IDXC = 25
ROWS_PER_TILE = N // NS


def _prep_body(feat_ref, mask_ref, w_ref, attn_ref, gh_ref, p_ref):
    fm = feat_ref[...] * mask_ref[...]
    h = lax.dot_general(fm, w_ref[...], (((1,), (1,)), ((), ())),
                        preferred_element_type=jnp.float32)
    s = jnp.dot(h, attn_ref[...], preferred_element_type=jnp.float32)
    t = jnp.where(s > 0.0, s, s * 0.01)
    p = jnp.exp(t - jnp.max(t))
    gh_ref[...] = h * p
    p_ref[...] = p.reshape(NR, L)


_prep = pl.pallas_call(
    _prep_body,
    out_shape=(jax.ShapeDtypeStruct((N, D), jnp.float32),
               jax.ShapeDtypeStruct((NR, L), jnp.float32)),
)


def _final_body(acc_ref, accp_ref, o_ref):
    num = acc_ref[0] + acc_ref[1]
    den = accp_ref[0] + accp_ref[1]
    o_ref[...] = jnp.maximum(num / jnp.where(den > 0.0, den, 1.0), 0.0)


_final = pl.pallas_call(
    _final_body,
    out_shape=jax.ShapeDtypeStruct((N, D), jnp.float32),
)


@functools.partial(
    pl.kernel,
    out_type=(jax.ShapeDtypeStruct((NC, N, D), jnp.float32),
              jax.ShapeDtypeStruct((NC, NR, L), jnp.float32)),
    mesh=plsc.VectorSubcoreMesh(core_axis_name="c", subcore_axis_name="s"),
    scratch_types=[
        pltpu.VMEM((IDXC, K), jnp.int32),
        pltpu.VMEM((IDXC, K), jnp.int32),
        pltpu.VMEM((K, D), jnp.float32),
        pltpu.VMEM((K, D), jnp.float32),
        pltpu.VMEM((NR, L), jnp.float32),
        pltpu.VMEM((NR, L), jnp.float32),
        pltpu.VMEM((5, 125), jnp.int32),
        pltpu.VMEM_SHARED((N, D), jnp.float32),
        pltpu.VMEM_SHARED((NR, L), jnp.float32),
        pltpu.SemaphoreType.DMA,
        pltpu.SemaphoreType.DMA,
        pltpu.SemaphoreType.DMA,
        pltpu.SemaphoreType.DMA,
    ],
    compiler_params=pltpu.CompilerParams(use_tc_tiling_on_sc=False,
                                         needs_layout_passes=False),
)
def _edge_kernel(g_hbm, p_hbm, e_hbm, iota_hbm, out_hbm, outp_hbm,
                 sidx, didx, a0, a1, pv, den, iota, acc, accsh,
                 gsem0, gsem1, ssem0, ssem1):
    cid = lax.axis_index("c")
    sid = lax.axis_index("s")
    w = cid * NS + sid

    pltpu.sync_copy(p_hbm, pv)
    pltpu.sync_copy(iota_hbm, iota)

    @pl.loop(0, NR)
    def _zero_den(i):
        den[i, pl.ds(0, L)] = jnp.zeros((16,), jnp.float32)

    @pl.loop(0, K)
    def _zero_rows(i):
        @pl.loop(0, D, step=16)
        def _zero_lane(c):
            a0[i, pl.ds(c, 16)] = jnp.zeros((16,), jnp.float32)

    @pl.loop(0, 560, step=K)
    def _zero_acc(r):
        pltpu.sync_copy(a0, acc.at[pl.ds(sid * ROWS_PER_TILE + r, K)])
    pltpu.sync_copy(a0.at[pl.ds(0, 65)],
                    acc.at[pl.ds(sid * ROWS_PER_TILE + 560, 65)])

    @pl.when(sid == 0)
    def _zero_accsh():
        @pl.loop(0, 5)
        def _z5(c):
            pltpu.sync_copy(den.at[pl.ds(0, 125)],
                            accsh.at[pl.ds(c * 125, 125)])

    plsc.subcore_barrier()

    def _gather(i, buf, sem):
        pltpu.async_copy(g_hbm.at[sidx.at[i]], buf, sem)

    def _gather_wait(i, buf, sem):
        pltpu.make_async_copy(g_hbm.at[sidx.at[i]], buf, sem).wait()

    @pl.loop(0, NB // IDXC)
    def _chunk(c):
        pltpu.sync_copy(e_hbm.at[0, w, pl.ds(c * IDXC, IDXC)], sidx)
        pltpu.sync_copy(e_hbm.at[1, w, pl.ds(c * IDXC, IDXC)], didx)

        _gather(0, a0, gsem0)
        _gather(1, a1, gsem1)

        @pl.loop(0, IDXC)
        def _denj(j):
            @pl.loop(0, K, step=16)
            def _denc(cc):
                sv = sidx[j, pl.ds(cc, 16)]
                dv = didx[j, pl.ds(cc, 16)]
                vals = plsc.load_gather(
                    pv, [lax.shift_right_logical(sv, 4),
                         lax.bitwise_and(sv, 15)])
                plsc.addupdate_scatter(
                    den, [lax.shift_right_logical(dv, 4),
                          lax.bitwise_and(dv, 15)], vals)

        @pl.loop(0, IDXC - 2, step=2)
        def _blk(i):
            _gather_wait(i, a0, gsem0)
            s0 = pltpu.async_copy(a0, acc.at[didx.at[i]], ssem0, add=True)
            _gather_wait(i + 1, a1, gsem1)
            s1 = pltpu.async_copy(a1, acc.at[didx.at[i + 1]], ssem1, add=True)
            s0.wait()

            @pl.when(i + 2 < IDXC)
            def _la0():
                _gather(i + 2, a0, gsem0)
            s1.wait()

            @pl.when(i + 3 < IDXC)
            def _la1():
                _gather(i + 3, a1, gsem1)

        _gather_wait(IDXC - 1, a0, gsem0)
        pltpu.sync_copy(a0, acc.at[didx.at[IDXC - 1]], add=True)

    @pl.loop(0, 5)
    def _merge(c):
        pltpu.sync_copy(den.at[pl.ds(c * 125, 125)],
                        accsh.at[iota.at[c]], add=True)

    plsc.subcore_barrier()
    row0 = sid * ROWS_PER_TILE
    pltpu.sync_copy(acc.at[pl.ds(row0, ROWS_PER_TILE)],
                    out_hbm.at[cid, pl.ds(row0, ROWS_PER_TILE)])

    @pl.when(sid == 0)
    def _out_den():
        pltpu.sync_copy(accsh, outp_hbm.at[cid])


def kernel(feat, edge_index, biclique_mask, W, attn_param):
    e4 = edge_index.astype(jnp.int32).reshape(2, NW, NB, K)
    iota = jnp.arange(NR, dtype=jnp.int32).reshape(5, 125)
    gh, p16 = _prep(feat, biclique_mask.reshape(1, D), W, attn_param)
    acc, accp = _edge_kernel(gh, p16, e4, iota)
    return _final(acc, accp.reshape(NC, N, 1))


# speedup vs baseline: 23.8890x; 1.0001x over previous
import functools

import jax
import jax.numpy as jnp
from jax import lax
from jax.experimental import pallas as pl
from jax.experimental.pallas import tpu as pltpu
from jax.experimental.pallas import tpu_sc as plsc

N = 10000
E = 320000
D = 128
L = 16
NR = N // L
NC = 2
NS = 16
NW = NC * NS
K = 80
NB = E // (NW * K)
IDXC = 25
ROWS_PER_TILE = N // NS


def _prep_body(feat_ref, mask_ref, w_ref, attn_ref, gh_ref, p_ref):
    fm = feat_ref[...] * mask_ref[...]
    h = lax.dot_general(fm, w_ref[...], (((1,), (1,)), ((), ())),
                        preferred_element_type=jnp.float32)
    s = jnp.dot(h, attn_ref[...], preferred_element_type=jnp.float32)
    t = jnp.where(s > 0.0, s, s * 0.01)
    p = jnp.exp(t - jnp.max(t))
    gh_ref[...] = h * p
    p_ref[...] = p.reshape(NR, L)


_prep = pl.pallas_call(
    _prep_body,
    out_shape=(jax.ShapeDtypeStruct((N, D), jnp.float32),
               jax.ShapeDtypeStruct((NR, L), jnp.float32)),
)


def _final_body(acc_ref, accp_ref, o_ref):
    num = acc_ref[0] + acc_ref[1]
    den = accp_ref[0] + accp_ref[1]
    o_ref[...] = jnp.maximum(num / jnp.where(den > 0.0, den, 1.0), 0.0)


_final = pl.pallas_call(
    _final_body,
    out_shape=jax.ShapeDtypeStruct((N, D), jnp.float32),
)


@functools.partial(
    pl.kernel,
    out_type=(jax.ShapeDtypeStruct((NC, N, D), jnp.float32),
              jax.ShapeDtypeStruct((NC, NR, L), jnp.float32)),
    mesh=plsc.VectorSubcoreMesh(core_axis_name="c", subcore_axis_name="s"),
    scratch_types=[
        pltpu.VMEM((IDXC, K), jnp.int32),
        pltpu.VMEM((IDXC, K), jnp.int32),
        pltpu.VMEM((K, D), jnp.float32),
        pltpu.VMEM((K, D), jnp.float32),
        pltpu.VMEM((NR, L), jnp.float32),
        pltpu.VMEM((NR, L), jnp.float32),
        pltpu.VMEM((5, 125), jnp.int32),
        pltpu.VMEM_SHARED((N, D), jnp.float32),
        pltpu.VMEM_SHARED((NR, L), jnp.float32),
        pltpu.SemaphoreType.DMA,
        pltpu.SemaphoreType.DMA,
        pltpu.SemaphoreType.DMA,
        pltpu.SemaphoreType.DMA,
    ],
    compiler_params=pltpu.CompilerParams(use_tc_tiling_on_sc=False,
                                         needs_layout_passes=False),
)
def _edge_kernel(g_hbm, p_hbm, e_hbm, iota_hbm, out_hbm, outp_hbm,
                 sidx, didx, a0, a1, pv, den, iota, acc, accsh,
                 gsem0, gsem1, ssem0, ssem1):
    cid = lax.axis_index("c")
    sid = lax.axis_index("s")
    w = cid * NS + sid

    pltpu.sync_copy(p_hbm, pv)
    pltpu.sync_copy(iota_hbm, iota)

    @pl.loop(0, NR)
    def _zero_den(i):
        den[i, pl.ds(0, L)] = jnp.zeros((16,), jnp.float32)

    @pl.loop(0, K)
    def _zero_rows(i):
        @pl.loop(0, D, step=16)
        def _zero_lane(c):
            a0[i, pl.ds(c, 16)] = jnp.zeros((16,), jnp.float32)

    @pl.loop(0, 560, step=K)
    def _zero_acc(r):
        pltpu.sync_copy(a0, acc.at[pl.ds(sid * ROWS_PER_TILE + r, K)])
    pltpu.sync_copy(a0.at[pl.ds(0, 65)],
                    acc.at[pl.ds(sid * ROWS_PER_TILE + 560, 65)])

    @pl.when(sid == 0)
    def _zero_accsh():
        @pl.loop(0, 5)
        def _z5(c):
            pltpu.sync_copy(den.at[pl.ds(0, 125)],
                            accsh.at[pl.ds(c * 125, 125)])

    plsc.subcore_barrier()

    def _gather(i, buf, sem):
        pltpu.async_copy(g_hbm.at[sidx.at[i]], buf, sem)

    def _gather_wait(i, buf, sem):
        pltpu.make_async_copy(g_hbm.at[sidx.at[i]], buf, sem).wait()

    @pl.loop(0, NB // IDXC)
    def _chunk(c):
        pltpu.sync_copy(e_hbm.at[0, w, pl.ds(c * IDXC, IDXC)], sidx)
        pltpu.sync_copy(e_hbm.at[1, w, pl.ds(c * IDXC, IDXC)], didx)

        _gather(0, a0, gsem0)
        _gather(1, a1, gsem1)

        def _den_sweep(j):
            @pl.loop(0, K, step=16)
            def _denc(cc):
                sv = sidx[j, pl.ds(cc, 16)]
                dv = didx[j, pl.ds(cc, 16)]
                vals = plsc.load_gather(
                    pv, [lax.shift_right_logical(sv, 4),
                         lax.bitwise_and(sv, 15)])
                plsc.addupdate_scatter(
                    den, [lax.shift_right_logical(dv, 4),
                          lax.bitwise_and(dv, 15)], vals)

        @pl.loop(0, IDXC - 2, step=2)
        def _blk(i):
            _gather_wait(i, a0, gsem0)
            s0 = pltpu.async_copy(a0, acc.at[didx.at[i]], ssem0, add=True)
            _gather_wait(i + 1, a1, gsem1)
            s1 = pltpu.async_copy(a1, acc.at[didx.at[i + 1]], ssem1, add=True)
            s0.wait()

            @pl.when(i + 2 < IDXC)
            def _la0():
                _gather(i + 2, a0, gsem0)
            s1.wait()

            @pl.when(i + 3 < IDXC)
            def _la1():
                _gather(i + 3, a1, gsem1)

            _den_sweep(i)
            _den_sweep(i + 1)

        _gather_wait(IDXC - 1, a0, gsem0)
        pltpu.sync_copy(a0, acc.at[didx.at[IDXC - 1]], add=True)
        _den_sweep(IDXC - 1)

    @pl.loop(0, 5)
    def _merge(c):
        pltpu.sync_copy(den.at[pl.ds(c * 125, 125)],
                        accsh.at[iota.at[c]], add=True)

    plsc.subcore_barrier()
    row0 = sid * ROWS_PER_TILE
    pltpu.sync_copy(acc.at[pl.ds(row0, ROWS_PER_TILE)],
                    out_hbm.at[cid, pl.ds(row0, ROWS_PER_TILE)])

    @pl.when(sid == 0)
    def _out_den():
        pltpu.sync_copy(accsh, outp_hbm.at[cid])


def kernel(feat, edge_index, biclique_mask, W, attn_param):
    e4 = edge_index.astype(jnp.int32).reshape(2, NW, NB, K)
    iota = jnp.arange(NR, dtype=jnp.int32).reshape(5, 125)
    gh, p16 = _prep(feat, biclique_mask.reshape(1, D), W, attn_param)
    acc, accp = _edge_kernel(gh, p16, e4, iota)
    return _final(acc, accp.reshape(NC, N, 1))
